# Initial kernel scaffold; baseline (speedup 1.0000x reference)
import jax
import jax.numpy as jnp
from jax.experimental import pallas as pl


def kernel(hidden_states, W, b):
    raise NotImplementedError("write your pallas kernel here")


# speedup vs baseline: 2.2913x; 2.2913x over previous
import jax
import jax.numpy as jnp
import numpy as np
from jax import lax
from jax.experimental import pallas as pl
from jax.experimental.pallas import tpu as pltpu

_T, _H, _E = 32768, 768, 64
_ITERS = 30
_BT = 2048
_NB = _T // _BT
_CH = 2048
_NCH = _T // _CH
_EPS = 1e-8


def _blk(ix, size):
    return pl.ds(pl.multiple_of(ix * np.int32(size), size), size)


def _router_kernel(x_ref, wt_ref, b_ref, logitsT_ref, affT_ref, idxT_ref,
                   costT_ref, d0_ref):
    i = pl.program_id(0)
    lg = lax.dot_general(wt_ref[...], x_ref[...], (((1,), (1,)), ((), ())),
                         preferred_element_type=jnp.float32)
    lg = lg + b_ref[...]
    logitsT_ref[...] = lg
    affT_ref[...] = jax.nn.sigmoid(lg)
    costT_ref[:, _blk(i, _BT)] = jnp.exp(lg)

    @pl.when(i == _NB - 1)
    def _phase2():
        def _ch(off):
            return pl.ds(pl.multiple_of(off, _CH), _CH)

        def iter_body(state):
            it, d1 = state

            def chunk_body(carry):
                off, acc = carry
                blk = costT_ref[:, _ch(off)]
                s0 = jnp.sum(blk * d1, axis=0, keepdims=True)
                d0 = (1.0 / _T) * (1.0 / (s0 + _EPS))
                d0_ref[:, _ch(off)] = d0
                colp = jnp.sum(blk * d0, axis=1, keepdims=True)
                return (off + np.int32(_CH), acc + colp)

            _, colsum = lax.while_loop(
                lambda c: c[0] < np.int32(_T), chunk_body,
                (np.int32(0), jnp.zeros((_E, 1), jnp.float32)))
            d1n = (1.0 / _E) * (1.0 / (colsum + _EPS))
            return (it + np.int32(1), d1n)

        _, d1 = lax.while_loop(lambda s: s[0] < np.int32(_ITERS), iter_body,
                               (np.int32(0), jnp.ones((_E, 1), jnp.float32)))

        def am_body(off):
            blk = costT_ref[:, _ch(off)]
            v = (d1 * blk) * d0_ref[:, _ch(off)]
            maxv = jnp.max(v, axis=0, keepdims=True)
            iota = lax.broadcasted_iota(jnp.int32, (_E, _CH), 0)
            idx = jnp.min(jnp.where(v == maxv, iota, np.int32(_E)), axis=0,
                          keepdims=True)
            idxT_ref[:, _ch(off)] = idx
            return off + np.int32(_CH)
        lax.while_loop(lambda o: o < np.int32(_T), am_body, np.int32(0))


def kernel(hidden_states, W, b):
    wt = W.T
    b2 = b.reshape(_E, 1).astype(jnp.float32)
    logitsT, affT, idxT = pl.pallas_call(
        _router_kernel,
        grid=(_NB,),
        in_specs=[
            pl.BlockSpec((_BT, _H), lambda i: (i, np.int32(0))),
            pl.BlockSpec((_E, _H), lambda i: (np.int32(0), np.int32(0))),
            pl.BlockSpec((_E, 1), lambda i: (np.int32(0), np.int32(0))),
        ],
        out_specs=[
            pl.BlockSpec((_E, _BT), lambda i: (np.int32(0), i)),
            pl.BlockSpec((_E, _BT), lambda i: (np.int32(0), i)),
            pl.BlockSpec((1, _T), lambda i: (np.int32(0), np.int32(0))),
        ],
        out_shape=[
            jax.ShapeDtypeStruct((_E, _T), jnp.float32),
            jax.ShapeDtypeStruct((_E, _T), jnp.float32),
            jax.ShapeDtypeStruct((1, _T), jnp.int32),
        ],
        scratch_shapes=[
            pltpu.VMEM((_E, _T), jnp.float32),
            pltpu.VMEM((1, _T), jnp.float32),
        ],
    )(hidden_states, wt, b2)
    router_logits = logitsT.T
    expert_affinities = affT.T
    expert_index = idxT.T.astype(jnp.int64)
    return (router_logits, expert_affinities, expert_index)
